# Initial kernel scaffold; baseline (speedup 1.0000x reference)
import functools

import jax
import jax.numpy as jnp
from jax import lax
from jax.experimental import pallas as pl
from jax.experimental.pallas import tpu as pltpu
from jax.experimental.pallas import tpu_sc as plsc

N_FIELDS = 26
VOCAB = 100000
EMB_DIM = 32

NC = 2
NS = 16
NW = NC * NS

CHUNK = 128
NBUF = 8


@functools.lru_cache(maxsize=None)
def _make_gather(n_rows: int):
    assert n_rows % (NW * CHUNK) == 0
    n_chunks = n_rows // (NW * CHUNK)
    assert n_chunks % NBUF == 0

    mesh = plsc.VectorSubcoreMesh(
        core_axis_name="c", subcore_axis_name="s", num_cores=NC, num_subcores=NS
    )

    def body(table, idx, out, idx_v, bufs, gsem, ssem):
        wid = lax.axis_index("s") * NC + lax.axis_index("c")
        base = wid * (n_chunks * CHUNK)
        pltpu.sync_copy(idx.at[wid], idx_v)

        def start_gather(j, b):
            pltpu.async_copy(table.at[idx_v.at[j]], bufs.at[b], gsem.at[b])

        def wait_gather(b):
            pltpu.make_async_copy(
                table.at[pl.ds(0, CHUNK)], bufs.at[b], gsem.at[b]
            ).wait()

        def start_store(j, b):
            pltpu.async_copy(
                bufs.at[b], out.at[pl.ds(base + j * CHUNK, CHUNK)], ssem.at[b]
            )

        def wait_store(b):
            pltpu.make_async_copy(
                bufs.at[b], out.at[pl.ds(0, CHUNK)], ssem.at[b]
            ).wait()

        for b in range(NBUF):
            start_gather(b, b)

        @pl.loop(0, n_chunks, step=NBUF)
        def _(g):
            for b in range(NBUF):
                j = g + b
                wait_gather(b)
                start_store(j, b)
                nj = j + NBUF

                @pl.when(nj < n_chunks)
                def _():
                    wait_store(b)
                    start_gather(nj, b)

        for b in range(NBUF):
            wait_store(b)

    return pl.kernel(
        body,
        out_type=jax.ShapeDtypeStruct((n_rows, EMB_DIM), jnp.float32),
        mesh=mesh,
        scratch_types=[
            pltpu.VMEM((n_chunks, CHUNK), jnp.int32),
            pltpu.VMEM((NBUF, CHUNK, EMB_DIM), jnp.float32),
            pltpu.SemaphoreType.DMA((NBUF,)),
            pltpu.SemaphoreType.DMA((NBUF,)),
        ],
    )


def kernel(cat_features, tables):
    batch = cat_features.shape[0]
    n_rows = batch * N_FIELDS
    cat = cat_features.astype(jnp.int32)
    offs = jnp.arange(N_FIELDS, dtype=jnp.int32) * VOCAB
    idx = (cat + offs[None, :]).reshape(NW, n_rows // (NW * CHUNK), CHUNK)
    flat_table = tables.reshape(N_FIELDS * VOCAB, EMB_DIM)
    out = _make_gather(n_rows)(flat_table, idx)
    return out.reshape(batch, N_FIELDS * EMB_DIM)


# speedup vs baseline: 1.2177x; 1.2177x over previous
import functools

import jax
import jax.numpy as jnp
from jax import lax
from jax.experimental import pallas as pl
from jax.experimental.pallas import tpu as pltpu
from jax.experimental.pallas import tpu_sc as plsc

N_FIELDS = 26
VOCAB = 100000
EMB_DIM = 32

NC = 2
NS = 16
NW = NC * NS

CHUNK = 128
NBUF = 8


@functools.lru_cache(maxsize=None)
def _make_gather(n_rows: int):
    assert n_rows % (NW * CHUNK) == 0
    n_chunks = n_rows // (NW * CHUNK)
    assert n_chunks % NBUF == 0

    mesh = plsc.VectorSubcoreMesh(
        core_axis_name="c", subcore_axis_name="s", num_cores=NC, num_subcores=NS
    )

    def body(table, idx, out, idx_v, bufs, gsem, ssem):
        wid = lax.axis_index("s") * NC + lax.axis_index("c")
        base = wid * (n_chunks * CHUNK)
        pltpu.sync_copy(idx.at[wid], idx_v)

        def start_gather(j, b):
            pltpu.async_copy(table.at[idx_v.at[j]], bufs.at[b], gsem.at[b])

        def wait_gather(b):
            pltpu.make_async_copy(
                table.at[pl.ds(0, CHUNK)], bufs.at[b], gsem.at[b]
            ).wait()

        def start_store(j, b):
            pltpu.async_copy(
                bufs.at[b], out.at[pl.ds(base + j * CHUNK, CHUNK)], ssem.at[b]
            )

        def wait_store(b):
            pltpu.make_async_copy(
                bufs.at[b], out.at[pl.ds(0, CHUNK)], ssem.at[b]
            ).wait()

        for b in range(NBUF):
            start_gather(b, b)

        @pl.loop(0, n_chunks, step=NBUF)
        def _(g):
            for b in range(NBUF):
                j = g + b
                wait_gather(b)
                start_store(j, b)
                nj = j + NBUF

                @pl.when(nj < n_chunks)
                def _():
                    wait_store(b)
                    start_gather(nj, b)

        for b in range(NBUF):
            wait_store(b)

    return pl.kernel(
        body,
        out_type=jax.ShapeDtypeStruct((n_rows, EMB_DIM), jnp.float32),
        mesh=mesh,
        compiler_params=pltpu.CompilerParams(use_tc_tiling_on_sc=False),
        scratch_types=[
            pltpu.VMEM((n_chunks, CHUNK), jnp.int32),
            pltpu.VMEM((NBUF, CHUNK, EMB_DIM), jnp.float32),
            pltpu.SemaphoreType.DMA((NBUF,)),
            pltpu.SemaphoreType.DMA((NBUF,)),
        ],
    )


def kernel(cat_features, tables):
    batch = cat_features.shape[0]
    n_rows = batch * N_FIELDS
    cat = cat_features.astype(jnp.int32)
    offs = jnp.arange(N_FIELDS, dtype=jnp.int32) * VOCAB
    idx = (cat + offs[None, :]).reshape(NW, n_rows // (NW * CHUNK), CHUNK)
    flat_table = tables.reshape(N_FIELDS * VOCAB, EMB_DIM)
    out = _make_gather(n_rows)(flat_table, idx)
    return out.reshape(batch, N_FIELDS * EMB_DIM)
